# baseline (speedup 1.0000x reference)
import functools

import jax
import jax.numpy as jnp
from jax import lax
from jax.experimental import pallas as pl
from jax.experimental.pallas import tpu as pltpu
from jax.experimental.pallas import tpu_sc as plsc

_NC = 2
_NS = 16
_NW = _NC * _NS
_LANES = 16
_CHUNK = 128
_NP = 10240

_SELU_SCALE = 1.0507009873554805
_SELU_ALPHA = 1.6732632423543772


def _sc_degree(adj2d, nchunks):
    maxc = adj2d.shape[1] // _NW
    mesh = plsc.VectorSubcoreMesh(core_axis_name="c", subcore_axis_name="s")

    @functools.partial(
        pl.kernel,
        mesh=mesh,
        out_type=jax.ShapeDtypeStruct((_NW, _NP), jnp.float32),
        scratch_types=[
            pltpu.VMEM((_NP,), jnp.float32),
            pltpu.VMEM((maxc, _CHUNK), jnp.int32),
        ],
        compiler_params=pltpu.CompilerParams(needs_layout_passes=False),
    )
    def k(adj_hbm, out_hbm, hist, idxb):
        c = lax.axis_index("c")
        s = lax.axis_index("s")
        wid = s * _NC + c
        c0 = pl.multiple_of(wid * maxc, 8)
        nch = jnp.minimum(jnp.maximum(nchunks - wid * maxc, 0), maxc)

        pltpu.sync_copy(adj_hbm.at[1, pl.ds(c0, maxc)], idxb)

        z16 = jnp.zeros((_LANES,), jnp.float32)

        def zbody(i, carry):
            base = i * 8 * _LANES
            for t in range(8):
                hist[pl.ds(base + t * _LANES, _LANES)] = z16
            return carry

        lax.fori_loop(0, _NP // (8 * _LANES), zbody, 0)

        ones16 = jnp.full((_LANES,), 1.0, jnp.float32)

        def body(j, carry):
            @pl.when(j < nch)
            def _():
                for t in range(_CHUNK // _LANES):
                    idx16 = idxb[j, pl.ds(t * _LANES, _LANES)]
                    plsc.addupdate_scatter(hist, [idx16], ones16)

            return carry

        lax.fori_loop(0, maxc, body, 0)
        pltpu.sync_copy(hist, out_hbm.at[wid])

    return k(adj2d)


_W = 16


def _sc_aggregate(hs, adj2d, nchunks):
    n, d = hs.shape
    maxc = adj2d.shape[1] // _NW
    rpt = _NP // _NS
    rlast = n - (_NS - 1) * rpt
    mesh = plsc.VectorSubcoreMesh(core_axis_name="c", subcore_axis_name="s")

    @functools.partial(
        pl.kernel,
        mesh=mesh,
        out_type=jax.ShapeDtypeStruct((_NC, _NP, d), jnp.float32),
        scratch_types=[
            pltpu.VMEM_SHARED((_NP, d), jnp.float32),
            pltpu.VMEM((2 * _W, _CHUNK), jnp.int32),
            pltpu.VMEM((2 * _W, _CHUNK), jnp.int32),
            pltpu.VMEM((_CHUNK, d), jnp.float32),
            pltpu.VMEM((_CHUNK, d), jnp.float32),
            pltpu.SemaphoreType.DMA,
            pltpu.SemaphoreType.DMA,
            pltpu.SemaphoreType.DMA,
            pltpu.SemaphoreType.DMA,
        ],
    )
    def k(hs_hbm, adj_hbm, out_hbm,
          acc, idx_s, idx_d, rows0, rows1, sem0, sem1, semis, semid):
        c = lax.axis_index("c")
        s = lax.axis_index("s")
        wid = s * _NC + c
        c0 = pl.multiple_of(wid * maxc, 8)
        nch = jnp.minimum(jnp.maximum(nchunks - wid * maxc, 0), maxc)
        r0 = pl.multiple_of(s * rpt, 8)

        @pl.when(jnp.logical_and(c == 0, s < _NS - 1))
        def _():
            pltpu.sync_copy(hs_hbm.at[pl.ds(r0, rpt)], acc.at[pl.ds(r0, rpt)])

        @pl.when(jnp.logical_and(c == 0, s == _NS - 1))
        def _():
            pltpu.sync_copy(hs_hbm.at[pl.ds(r0, rlast)], acc.at[pl.ds(r0, rlast)])

        @pl.when(c == 1)
        def _():
            z16 = jnp.zeros((_LANES,), jnp.float32)

            def zb(r, carry):
                for t in range(d // _LANES):
                    rows0[r, pl.ds(t * _LANES, _LANES)] = z16
                return carry

            lax.fori_loop(0, _CHUNK, zb, 0)
            for p in range(rpt // _CHUNK):
                pltpu.sync_copy(
                    rows0, acc.at[pl.ds(pl.multiple_of(r0 + p * _CHUNK, 8),
                                        _CHUNK)])

        plsc.subcore_barrier()

        bufs = ((rows0, sem0), (rows1, sem1))
        nwin = maxc // _W

        pltpu.sync_copy(adj_hbm.at[0, pl.ds(c0, _W)], idx_s.at[pl.ds(0, _W)])
        pltpu.sync_copy(adj_hbm.at[1, pl.ds(c0, _W)], idx_d.at[pl.ds(0, _W)])

        @pl.when(0 < nch)
        def _():
            pltpu.make_async_copy(hs_hbm.at[idx_s.at[0]], rows0, sem0).start()

        def win_body(w, carry):
            par = w % 2
            off = par * _W
            offn = _W - off
            j0 = w * _W
            nb = pl.multiple_of(c0 + (w + 1) * _W, 8)

            @pl.when(w + 1 < nwin)
            def _():
                pltpu.make_async_copy(
                    adj_hbm.at[0, pl.ds(nb, _W)],
                    idx_s.at[pl.ds(offn, _W)], semis).start()
                pltpu.make_async_copy(
                    adj_hbm.at[1, pl.ds(nb, _W)],
                    idx_d.at[pl.ds(offn, _W)], semid).start()

            def pair_body(t2, carry2):
                @pl.when(jnp.logical_and(t2 == 3, w + 1 < nwin))
                def _():
                    pltpu.make_async_copy(
                        adj_hbm.at[0, pl.ds(nb, _W)],
                        idx_s.at[pl.ds(offn, _W)], semis).wait()
                    pltpu.make_async_copy(
                        adj_hbm.at[1, pl.ds(nb, _W)],
                        idx_d.at[pl.ds(offn, _W)], semid).wait()

                for b in range(2):
                    t = t2 * 2 + b
                    j = j0 + t
                    rb, sb = bufs[b]
                    rn, sn = bufs[1 - b]
                    nxt_row = jnp.where(t + 1 < _W, off + t + 1, offn)

                    @pl.when(j + 1 < nch)
                    def _():
                        pltpu.make_async_copy(
                            hs_hbm.at[idx_s.at[nxt_row]], rn, sn).start()

                    @pl.when(j < nch)
                    def _():
                        pltpu.make_async_copy(
                            hs_hbm.at[idx_s.at[off + t]], rb, sb).wait()
                        pltpu.sync_copy(rb, acc.at[idx_d.at[off + t]], add=True)

                return carry2

            lax.fori_loop(0, _W // 2, pair_body, 0)
            return carry

        lax.fori_loop(0, nwin, win_body, 0)
        plsc.subcore_barrier()
        pltpu.sync_copy(acc.at[pl.ds(r0, rpt)], out_hbm.at[c, pl.ds(r0, rpt)])

    return k(hs, adj2d)


def _tc_scale_matmul(x, w, degp):
    n, d = x.shape
    nw = degp.shape[0]
    blk = 2048

    def body(x_ref, w_ref, deg_ref, o_ref, dis_ref):
        deg = jnp.sum(deg_ref[...], axis=0) + 1.0
        disv = lax.rsqrt(deg).reshape(blk, 1)
        dis_ref[...] = disv
        h = jnp.dot(x_ref[...], w_ref[...], preferred_element_type=jnp.float32)
        o_ref[...] = h * disv

    return pl.pallas_call(
        body,
        grid=(-(-n // blk),),
        in_specs=[
            pl.BlockSpec((blk, d), lambda i: (i, 0)),
            pl.BlockSpec((d, d), lambda i: (0, 0)),
            pl.BlockSpec((nw, blk), lambda i: (0, i)),
        ],
        out_specs=[
            pl.BlockSpec((blk, d), lambda i: (i, 0)),
            pl.BlockSpec((blk, 1), lambda i: (i, 0)),
        ],
        out_shape=[
            jax.ShapeDtypeStruct((n, d), jnp.float32),
            jax.ShapeDtypeStruct((n, 1), jnp.float32),
        ],
    )(x, w, degp)


def _tc_combine_selu_matmul(y, n, dis, b, w):
    d = y.shape[2]
    blk = 2000

    def body(y_ref, dis_ref, b_ref, w_ref, o_ref):
        t = dis_ref[...] * (y_ref[0] + y_ref[1]) + b_ref[...]
        a = _SELU_SCALE * jnp.where(t > 0, t, _SELU_ALPHA * (jnp.exp(t) - 1.0))
        h = jnp.dot(a, w_ref[...], preferred_element_type=jnp.float32)
        o_ref[...] = h * dis_ref[...]

    return pl.pallas_call(
        body,
        grid=(n // blk,),
        in_specs=[
            pl.BlockSpec((_NC, blk, d), lambda i: (0, i, 0)),
            pl.BlockSpec((blk, 1), lambda i: (i, 0)),
            pl.BlockSpec((d,), lambda i: (0,)),
            pl.BlockSpec((d, d), lambda i: (0, 0)),
        ],
        out_specs=pl.BlockSpec((blk, d), lambda i: (i, 0)),
        out_shape=jax.ShapeDtypeStruct((n, d), jnp.float32),
    )(y, dis, b, w)


def _tc_combine(y, n, dis, b):
    d = y.shape[2]
    blk = 2000

    def body(y_ref, dis_ref, b_ref, o_ref):
        o_ref[...] = dis_ref[...] * (y_ref[0] + y_ref[1]) + b_ref[...]

    return pl.pallas_call(
        body,
        grid=(n // blk,),
        in_specs=[
            pl.BlockSpec((_NC, blk, d), lambda i: (0, i, 0)),
            pl.BlockSpec((blk, 1), lambda i: (i, 0)),
            pl.BlockSpec((d,), lambda i: (0,)),
        ],
        out_specs=pl.BlockSpec((blk, d), lambda i: (i, 0)),
        out_shape=jax.ShapeDtypeStruct((n, d), jnp.float32),
    )(y, dis, b)


def kernel(x, adj_t, W1, b1, W2, b2):
    n, d = x.shape
    e = adj_t.shape[1]
    nchunks = e // _CHUNK
    maxc = -(-nchunks // _NW)
    maxc = -(-maxc // _W) * _W

    adj2d = jnp.pad(adj_t.reshape(2, nchunks, _CHUNK),
                    ((0, 0), (0, _NW * maxc - nchunks), (0, 0)))
    degp = _sc_degree(adj2d, nchunks)

    hs1, dis = _tc_scale_matmul(x, W1, degp)
    y1 = _sc_aggregate(hs1, adj2d, nchunks)
    hs2 = _tc_combine_selu_matmul(y1, n, dis, b1, W2)
    y2 = _sc_aggregate(hs2, adj2d, nchunks)
    return _tc_combine(y2, n, dis, b2)
